# baseline (speedup 1.0000x reference)
import functools

import jax
import jax.numpy as jnp
from jax import lax
from jax.experimental import pallas as pl
from jax.experimental.pallas import tpu as pltpu
from jax.experimental.pallas import tpu_sc as plsc

VOCAB = 1000
EMBED = 4
BATCH = 1024
HIST = 50
NTOK = BATCH * HIST

NC, NS = 2, 16
NW = NC * NS
B_PER_W = NTOK // NW
CHUNK = 80
N_CHUNKS = B_PER_W // CHUNK


def _proj_body(table_ref, w_ref, b_ref, p_ref):
    p = lax.dot_general(
        table_ref[...], w_ref[...],
        dimension_numbers=(((1,), (1,)), ((), ())),
        preferred_element_type=jnp.float32,
    )
    p_ref[...] = p + b_ref[...]


def _compute_p(table, W, b):
    return pl.pallas_call(
        _proj_body,
        out_shape=jax.ShapeDtypeStruct((VOCAB, VOCAB), jnp.float32),
    )(table, W, b.reshape(1, VOCAB))


def _gather_body(p_hbm, x_hbm, out_hbm, idx_v, rows_v, sem):
    wid = lax.axis_index("s") * NC + lax.axis_index("c")
    base = wid * B_PER_W
    pltpu.sync_copy(x_hbm.at[pl.ds(base, B_PER_W)], idx_v)

    def chunk(g, _):
        idx_c = idx_v.at[pl.ds(g * CHUNK, CHUNK)]
        pltpu.async_copy(p_hbm.at[idx_c], rows_v, sem).wait()
        pltpu.sync_copy(rows_v, out_hbm.at[pl.ds(base + g * CHUNK, CHUNK)])
        return 0

    lax.fori_loop(0, N_CHUNKS, chunk, 0)


_gather = functools.partial(
    pl.kernel,
    mesh=plsc.VectorSubcoreMesh(core_axis_name="c", subcore_axis_name="s"),
    out_type=jax.ShapeDtypeStruct((NTOK, VOCAB), jnp.float32),
    compiler_params=pltpu.CompilerParams(use_tc_tiling_on_sc=False),
    scratch_types=[
        pltpu.VMEM((B_PER_W,), jnp.int32),
        pltpu.VMEM((CHUNK, VOCAB), jnp.float32),
        pltpu.SemaphoreType.DMA,
    ],
)(_gather_body)


def kernel(x, table, W, b):
    p = _compute_p(table, W, b)
    x_flat = x.reshape(NTOK).astype(jnp.int32)
    out = _gather(p, x_flat)
    return out.reshape(BATCH, HIST, VOCAB)
